# baseline (device time: 78611 ns/iter reference)
import jax
import jax.numpy as jnp
from jax import lax
from jax.experimental import pallas as pl
from jax.experimental.pallas import tpu as pltpu

N_DEV = 16


def kernel(A, B):
    m, k = A.shape
    k2, n = B.shape
    assert k == k2
    chunk = m // N_DEV

    def body(a_ref, b_ref, out_ref, p_ref, comm_ref, send_sems, recv_sems):
        my = lax.axis_index("i")
        left = (my - 1) % N_DEV
        right = (my + 1) % N_DEV

        barrier_sem = pltpu.get_barrier_semaphore()
        for nbr in (left, right):
            pl.semaphore_signal(
                barrier_sem, inc=1,
                device_id=(nbr,), device_id_type=pl.DeviceIdType.MESH,
            )
        pl.semaphore_wait(barrier_sem, 2)

        p_ref[:, :] = jnp.dot(
            a_ref[:, :].astype(jnp.bfloat16),
            b_ref[:, :].astype(jnp.bfloat16),
            preferred_element_type=jnp.float32,
        )

        c0 = (my - 1) % N_DEV
        comm_ref[0, :, :] = p_ref[pl.ds(c0 * chunk, chunk), :]

        for h in range(N_DEV - 1):
            rdma = pltpu.make_async_remote_copy(
                src_ref=comm_ref.at[h],
                dst_ref=comm_ref.at[h + 1],
                send_sem=send_sems.at[h],
                recv_sem=recv_sems.at[h + 1],
                device_id=(right,),
                device_id_type=pl.DeviceIdType.MESH,
            )
            rdma.start()
            rdma.wait()
            c = (my - h - 2) % N_DEV
            comm_ref[h + 1, :, :] = (
                comm_ref[h + 1, :, :] + p_ref[pl.ds(c * chunk, chunk), :]
            )

        out_ref[:, :] = comm_ref[N_DEV - 1, :, :]

    return pl.pallas_call(
        body,
        out_shape=jax.ShapeDtypeStruct((chunk, n), jnp.float32),
        in_specs=[
            pl.BlockSpec(memory_space=pltpu.VMEM),
            pl.BlockSpec(memory_space=pltpu.VMEM),
        ],
        out_specs=pl.BlockSpec(memory_space=pltpu.VMEM),
        scratch_shapes=[
            pltpu.VMEM((m, n), jnp.float32),
            pltpu.VMEM((N_DEV, chunk, n), jnp.float32),
            pltpu.SemaphoreType.DMA((N_DEV,)),
            pltpu.SemaphoreType.DMA((N_DEV,)),
        ],
        compiler_params=pltpu.CompilerParams(collective_id=0),
    )(A, B)


# device time: 30602 ns/iter; 2.5688x vs baseline; 2.5688x over previous
import jax
import jax.numpy as jnp
from jax import lax
from jax.experimental import pallas as pl
from jax.experimental.pallas import tpu as pltpu

N_DEV = 16


def kernel(A, B):
    m, k = A.shape
    k2, n = B.shape
    assert k == k2
    chunk = m // N_DEV

    def body(a_ref, b_ref, out_ref, p_ref, send_ref, recv_ref,
             send_sems, recv_sems):
        my = lax.axis_index("i")

        barrier_sem = pltpu.get_barrier_semaphore()
        for s in range(1, N_DEV):
            peer = (my + s) % N_DEV
            pl.semaphore_signal(
                barrier_sem, inc=1,
                device_id=(peer,), device_id_type=pl.DeviceIdType.MESH,
            )
        pl.semaphore_wait(barrier_sem, N_DEV - 1)

        p_ref[:, :] = jnp.dot(
            a_ref[:, :].astype(jnp.bfloat16),
            b_ref[:, :].astype(jnp.bfloat16),
            preferred_element_type=jnp.float32,
        )

        for s in range(1, N_DEV):
            t = (my + s) % N_DEV
            send_ref[s, :, :] = p_ref[pl.ds(t * chunk, chunk), :].astype(
                jnp.bfloat16
            )

        rdmas = []
        for s in range(1, N_DEV):
            t = (my + s) % N_DEV
            rdma = pltpu.make_async_remote_copy(
                src_ref=send_ref.at[s],
                dst_ref=recv_ref.at[N_DEV - s],
                send_sem=send_sems.at[s],
                recv_sem=recv_sems.at[N_DEV - s],
                device_id=(t,),
                device_id_type=pl.DeviceIdType.MESH,
            )
            rdma.start()
            rdmas.append(rdma)

        out_ref[:, :] = p_ref[pl.ds(my * chunk, chunk), :]

        for s, rdma in zip(range(1, N_DEV), rdmas):
            rdma.wait_recv()
            out_ref[:, :] = out_ref[:, :] + recv_ref[N_DEV - s, :, :].astype(
                jnp.float32
            )

        for rdma in rdmas:
            rdma.wait_send()

    return pl.pallas_call(
        body,
        out_shape=jax.ShapeDtypeStruct((chunk, n), jnp.float32),
        in_specs=[
            pl.BlockSpec(memory_space=pltpu.VMEM),
            pl.BlockSpec(memory_space=pltpu.VMEM),
        ],
        out_specs=pl.BlockSpec(memory_space=pltpu.VMEM),
        scratch_shapes=[
            pltpu.VMEM((m, n), jnp.float32),
            pltpu.VMEM((N_DEV, chunk, n), jnp.bfloat16),
            pltpu.VMEM((N_DEV, chunk, n), jnp.bfloat16),
            pltpu.SemaphoreType.DMA((N_DEV,)),
            pltpu.SemaphoreType.DMA((N_DEV,)),
        ],
        compiler_params=pltpu.CompilerParams(collective_id=0),
    )(A, B)


# device time: 29559 ns/iter; 2.6595x vs baseline; 1.0353x over previous
import jax
import jax.numpy as jnp
from jax import lax
from jax.experimental import pallas as pl
from jax.experimental.pallas import tpu as pltpu

N_DEV = 16
BLOCKS = 4


def kernel(A, B):
    m, k = A.shape
    k2, n = B.shape
    assert k == k2
    chunk = m // N_DEV
    stripe = m // BLOCKS
    per_stripe = N_DEV // BLOCKS

    def body(a_ref, b_ref, out_ref, b16_ref, send_ref, recv_ref,
             send_sems, recv_sems):
        my = lax.axis_index("i")

        barrier_sem = pltpu.get_barrier_semaphore()
        for s in range(1, N_DEV):
            peer = (my + s) % N_DEV
            pl.semaphore_signal(
                barrier_sem, inc=1,
                device_id=(peer,), device_id_type=pl.DeviceIdType.MESH,
            )
        pl.semaphore_wait(barrier_sem, N_DEV - 1)

        b16_ref[:, :] = b_ref[:, :].astype(jnp.bfloat16)

        def make_rdma(r):
            return pltpu.make_async_remote_copy(
                src_ref=send_ref.at[r],
                dst_ref=recv_ref.at[my],
                send_sem=send_sems.at[r],
                recv_sem=recv_sems.at[my],
                device_id=(r,),
                device_id_type=pl.DeviceIdType.MESH,
            )

        send_rdmas = {}
        for g in range(BLOCKS):
            blk = jnp.dot(
                a_ref[g * stripe:(g + 1) * stripe, :].astype(jnp.bfloat16),
                b16_ref[:, :],
                preferred_element_type=jnp.float32,
            )
            blk16 = blk.astype(jnp.bfloat16)
            for j in range(per_stripe):
                r = g * per_stripe + j
                send_ref[r, :, :] = blk16[j * chunk:(j + 1) * chunk, :]

                @pl.when(r == my)
                def _(blk=blk, j=j):
                    out_ref[:, :] = blk[j * chunk:(j + 1) * chunk, :]

                rdma = make_rdma(r)

                @pl.when(r != my)
                def _(rdma=rdma):
                    rdma.start()

                send_rdmas[r] = rdma

        for r in range(N_DEV):
            recv = pltpu.make_async_remote_copy(
                src_ref=send_ref.at[r],
                dst_ref=recv_ref.at[r],
                send_sem=send_sems.at[r],
                recv_sem=recv_sems.at[r],
                device_id=(r,),
                device_id_type=pl.DeviceIdType.MESH,
            )

            @pl.when(r != my)
            def _(recv=recv, r=r):
                recv.wait_recv()
                out_ref[:, :] = out_ref[:, :] + recv_ref[r, :, :].astype(
                    jnp.float32
                )

        for r in range(N_DEV):
            @pl.when(r != my)
            def _(rdma=send_rdmas[r]):
                rdma.wait_send()

    return pl.pallas_call(
        body,
        out_shape=jax.ShapeDtypeStruct((chunk, n), jnp.float32),
        in_specs=[
            pl.BlockSpec(memory_space=pltpu.VMEM),
            pl.BlockSpec(memory_space=pltpu.VMEM),
        ],
        out_specs=pl.BlockSpec(memory_space=pltpu.VMEM),
        scratch_shapes=[
            pltpu.VMEM((k, n), jnp.bfloat16),
            pltpu.VMEM((N_DEV, chunk, n), jnp.bfloat16),
            pltpu.VMEM((N_DEV, chunk, n), jnp.bfloat16),
            pltpu.SemaphoreType.DMA((N_DEV,)),
            pltpu.SemaphoreType.DMA((N_DEV,)),
        ],
        compiler_params=pltpu.CompilerParams(collective_id=0),
    )(A, B)


# device time: 29205 ns/iter; 2.6917x vs baseline; 1.0121x over previous
import jax
import jax.numpy as jnp
from jax import lax
from jax.experimental import pallas as pl
from jax.experimental.pallas import tpu as pltpu

N_DEV = 16
N_PLANES = 4
PER_PLANE = 4


def kernel(A, B):
    m, k = A.shape
    k2, n = B.shape
    assert k == k2
    chunk = m // N_DEV
    stripe = m // N_PLANES

    def body(a_ref, b_ref, out_ref, b16_ref, send_ref, recv_ref,
             send_sems, recv_sems):
        my = lax.axis_index("i")
        my_plane = my // PER_PLANE

        barrier_sem = pltpu.get_barrier_semaphore()
        for s in range(1, N_DEV):
            peer = (my + s) % N_DEV
            pl.semaphore_signal(
                barrier_sem, inc=1,
                device_id=(peer,), device_id_type=pl.DeviceIdType.MESH,
            )

        b16_ref[:, :] = b_ref[:, :].astype(jnp.bfloat16)

        send_rdmas = []
        for t in range(N_PLANES):
            tp = (my_plane + 1 + t) % N_PLANES
            blk = jnp.dot(
                a_ref[pl.ds(tp * stripe, stripe), :].astype(jnp.bfloat16),
                b16_ref[:, :],
                preferred_element_type=jnp.float32,
            )
            blk16 = blk.astype(jnp.bfloat16)
            for j in range(PER_PLANE):
                slot = t * PER_PLANE + j
                rid = tp * PER_PLANE + j
                send_ref[slot, :, :] = blk16[j * chunk:(j + 1) * chunk, :]

                @pl.when(rid == my)
                def _(blk=blk, j=j):
                    out_ref[:, :] = blk[j * chunk:(j + 1) * chunk, :]

                if t == 0 and j == 0:
                    pl.semaphore_wait(barrier_sem, N_DEV - 1)

                rdma = pltpu.make_async_remote_copy(
                    src_ref=send_ref.at[slot],
                    dst_ref=recv_ref.at[my],
                    send_sem=send_sems.at[slot],
                    recv_sem=recv_sems.at[my],
                    device_id=(rid,),
                    device_id_type=pl.DeviceIdType.MESH,
                )

                @pl.when(rid != my)
                def _(rdma=rdma):
                    rdma.start()

                send_rdmas.append(rdma)

        for t in range(N_PLANES):
            sp = (my_plane - 1 - t) % N_PLANES
            for c in range(PER_PLANE):
                sid = sp * PER_PLANE + c
                recv = pltpu.make_async_remote_copy(
                    src_ref=send_ref.at[t * PER_PLANE + c],
                    dst_ref=recv_ref.at[sid],
                    send_sem=send_sems.at[t * PER_PLANE + c],
                    recv_sem=recv_sems.at[sid],
                    device_id=(sid,),
                    device_id_type=pl.DeviceIdType.MESH,
                )

                @pl.when(sid != my)
                def _(recv=recv, sid=sid):
                    recv.wait_recv()
                    out_ref[:, :] = out_ref[:, :] + recv_ref[
                        sid, :, :
                    ].astype(jnp.float32)

        for i, rdma in enumerate(send_rdmas):
            tp = (my_plane + 1 + i // PER_PLANE) % N_PLANES
            rid = tp * PER_PLANE + (i % PER_PLANE)

            @pl.when(rid != my)
            def _(rdma=rdma):
                rdma.wait_send()

    return pl.pallas_call(
        body,
        out_shape=jax.ShapeDtypeStruct((chunk, n), jnp.float32),
        in_specs=[
            pl.BlockSpec(memory_space=pltpu.VMEM),
            pl.BlockSpec(memory_space=pltpu.VMEM),
        ],
        out_specs=pl.BlockSpec(memory_space=pltpu.VMEM),
        scratch_shapes=[
            pltpu.VMEM((k, n), jnp.bfloat16),
            pltpu.VMEM((N_DEV, chunk, n), jnp.bfloat16),
            pltpu.VMEM((N_DEV, chunk, n), jnp.bfloat16),
            pltpu.SemaphoreType.DMA((N_DEV,)),
            pltpu.SemaphoreType.DMA((N_DEV,)),
        ],
        compiler_params=pltpu.CompilerParams(collective_id=0),
    )(A, B)
